# baseline (speedup 1.0000x reference)
import functools

import jax
import jax.numpy as jnp
from jax import lax
from jax.experimental import pallas as pl
from jax.experimental.pallas import tpu as pltpu
from jax.experimental.pallas import tpu_sc as plsc

N = 10000
D = 128
HD = D // 2
E = 320000
NC, NS = 2, 16
NW = NC * NS
CH = 128
NCHUNK = 80
EPT = NCHUNK * CH
EPAD = NW * EPT
NPAD = 10112
OPT = NPAD // NS
NFULL = OPT // CH
NREM = OPT - NFULL * CH
NBUF = 4

_mesh = plsc.VectorSubcoreMesh(core_axis_name="c", subcore_axis_name="s")


@functools.partial(
    pl.kernel,
    out_type=[
        jax.ShapeDtypeStruct((2 * NC * NPAD, HD), jnp.float32),
        jax.ShapeDtypeStruct((NC * NPAD,), jnp.float32),
    ],
    mesh=_mesh,
    scratch_types=[
        pltpu.VMEM((NCHUNK, CH), jnp.int32),
        pltpu.VMEM((NCHUNK, CH), jnp.int32),
        pltpu.VMEM((NBUF, CH, HD), jnp.float32),
        pltpu.VMEM((CH,), jnp.float32),
        pltpu.VMEM((OPT,), jnp.float32),
        pltpu.VMEM_SHARED((NPAD, HD), jnp.float32),
        pltpu.VMEM_SHARED((NPAD,), jnp.float32),
        pltpu.SemaphoreType.DMA((NBUF,)),
        pltpu.SemaphoreType.DMA((NBUF,)),
        pltpu.SemaphoreType.DMA((NBUF,)),
    ],
    compiler_params=pltpu.CompilerParams(use_tc_tiling_on_sc=False),
)
def _phase1(x0_hbm, x1_hbm, col_hbm, row_hbm, xsum_out, deg_out,
            colv, rowv, rows, onesv, buf1, xsum_sh, deg_sh,
            gsem, ssem, dsem):
    c = lax.axis_index("c")
    s = lax.axis_index("s")
    wid = c * NS + s

    zero16 = jnp.zeros((16,), jnp.float32)
    one16 = jnp.ones((16,), jnp.float32)

    def _z1(i, _):
        buf1[pl.ds(i * 16, 16)] = zero16
        return 0
    lax.fori_loop(0, OPT // 16, _z1, 0)
    buf1[pl.ds(OPT - 16, 16)] = zero16

    def _o1(i, _):
        onesv[pl.ds(i * 16, 16)] = one16
        return 0
    lax.fori_loop(0, CH // 16, _o1, 0)

    pltpu.sync_copy(col_hbm.at[wid], colv)
    pltpu.sync_copy(row_hbm.at[wid], rowv)

    for h in range(2):
        x_hbm = x0_hbm if h == 0 else x1_hbm

        def _zrow(i, _):
            for j in range(HD // 16):
                rows[0, i, pl.ds(j * 16, 16)] = zero16
            return 0
        lax.fori_loop(0, CH, _zrow, 0)

        for k in range(NFULL):
            pltpu.sync_copy(rows.at[0],
                            xsum_sh.at[pl.ds(s * OPT + k * CH, CH)])
        pltpu.sync_copy(rows.at[0].at[pl.ds(0, NREM)],
                        xsum_sh.at[pl.ds(s * OPT + NFULL * CH, NREM)])
        if h == 0:
            pltpu.sync_copy(buf1, deg_sh.at[pl.ds(s * OPT, OPT)])
        plsc.subcore_barrier()

        def _gather(i, b):
            return pltpu.make_async_copy(
                x_hbm.at[colv.at[i]], rows.at[b], gsem.at[b])

        def _scat_rows(i, b):
            return pltpu.make_async_copy(
                rows.at[b], xsum_sh.at[rowv.at[i]], ssem.at[b])

        def _scat_deg(i, b):
            return pltpu.make_async_copy(
                onesv, deg_sh.at[rowv.at[i]], dsem.at[b])

        for b in range(NBUF):
            pltpu.async_copy(x_hbm.at[colv.at[b]], rows.at[b], gsem.at[b])

        def _outer(g, _):
            for b in range(NBUF):
                i = g * NBUF + b
                _gather(i, b).wait()

                if h == 0:
                    @pl.when(g > 0)
                    def _():
                        _scat_deg(i, b).wait()

                pltpu.async_copy(rows.at[b], xsum_sh.at[rowv.at[i]],
                                 ssem.at[b], add=True)
                if h == 0:
                    pltpu.async_copy(onesv, deg_sh.at[rowv.at[i]],
                                     dsem.at[b], add=True)
                _scat_rows(i, b).wait()

                @pl.when(i + NBUF < NCHUNK)
                def _():
                    pltpu.async_copy(x_hbm.at[colv.at[i + NBUF]],
                                     rows.at[b], gsem.at[b])
            return 0
        lax.fori_loop(0, NCHUNK // NBUF, _outer, 0)

        if h == 0:
            for b in range(NBUF):
                _scat_deg(0, b).wait()

        plsc.subcore_barrier()

        obase = (h * NC + c) * NPAD
        for k in range(NFULL):
            off = s * OPT + k * CH
            pltpu.sync_copy(xsum_sh.at[pl.ds(off, CH)], rows.at[0])
            pltpu.sync_copy(rows.at[0], xsum_out.at[pl.ds(obase + off, CH)])
        offr = s * OPT + NFULL * CH
        pltpu.sync_copy(xsum_sh.at[pl.ds(offr, NREM)],
                        rows.at[0].at[pl.ds(0, NREM)])
        pltpu.sync_copy(rows.at[0].at[pl.ds(0, NREM)],
                        xsum_out.at[pl.ds(obase + offr, NREM)])
        if h == 0:
            pltpu.sync_copy(deg_sh.at[pl.ds(s * OPT, OPT)], buf1)
            pltpu.sync_copy(buf1, deg_out.at[pl.ds(c * NPAD + s * OPT, OPT)])
            plsc.subcore_barrier()


def _phase2_body(x_ref, xs00_ref, xs01_ref, xs10_ref, xs11_ref,
                 degp_ref, w_ref, t_ref, s13_ref, recip_ref):
    x = x_ref[...]
    xsum = jnp.concatenate([xs00_ref[...] + xs01_ref[...],
                            xs10_ref[...] + xs11_ref[...]], axis=1)
    deg = degp_ref[0, :] + degp_ref[1, :]
    recip = 1.0 / jnp.maximum(deg, 1.0)
    w1 = w_ref[0:1, :]
    w2 = w_ref[1:2, :]
    w3 = w_ref[2:3, :]
    xmean = xsum * recip[:, None]
    t_ref[0, :] = jnp.sum(jnp.abs(x - xmean) * w2, axis=1)
    s13_ref[0, :] = (recip * jnp.sum(xsum * x * w1, axis=1)
                     + jnp.sum(x * w3, axis=1))
    recip_ref[0, :] = recip


_phase2 = pl.pallas_call(
    _phase2_body,
    out_shape=[jax.ShapeDtypeStruct((1, N), jnp.float32)] * 3,
)

NBUF3 = 8


@functools.partial(
    pl.kernel,
    out_type=[jax.ShapeDtypeStruct((NC * NPAD,), jnp.float32)],
    mesh=_mesh,
    scratch_types=[
        pltpu.VMEM((NCHUNK, CH), jnp.int32),
        pltpu.VMEM((NCHUNK, CH), jnp.int32),
        pltpu.VMEM((NCHUNK, CH), jnp.float32),
        pltpu.VMEM((OPT,), jnp.float32),
        pltpu.VMEM_SHARED((NPAD,), jnp.float32),
        pltpu.SemaphoreType.DMA((NBUF3,)),
        pltpu.SemaphoreType.DMA((NBUF3,)),
    ],
)
def _phase3(t_hbm, col_hbm, row_hbm, usum_out,
            colv, rowv, tvals, buf1, usum_sh, gsem, ssem):
    c = lax.axis_index("c")
    s = lax.axis_index("s")
    wid = c * NS + s

    zero16 = jnp.zeros((16,), jnp.float32)

    def _z1(i, _):
        buf1[pl.ds(i * 16, 16)] = zero16
        return 0
    lax.fori_loop(0, OPT // 16, _z1, 0)
    buf1[pl.ds(OPT - 16, 16)] = zero16
    pltpu.sync_copy(buf1, usum_sh.at[pl.ds(s * OPT, OPT)])

    pltpu.sync_copy(col_hbm.at[wid], colv)
    pltpu.sync_copy(row_hbm.at[wid], rowv)
    plsc.subcore_barrier()

    def _gather(i, b):
        return pltpu.make_async_copy(
            t_hbm.at[colv.at[i]], tvals.at[i], gsem.at[b])

    def _scat(i, b):
        return pltpu.make_async_copy(
            tvals.at[i], usum_sh.at[rowv.at[i]], ssem.at[b])

    for b in range(NBUF3):
        pltpu.async_copy(t_hbm.at[colv.at[b]], tvals.at[b], gsem.at[b])

    def _outer(g, _):
        for b in range(NBUF3):
            i = g * NBUF3 + b
            _gather(i, b).wait()

            @pl.when(g > 0)
            def _():
                _scat(i, b).wait()

            pltpu.async_copy(tvals.at[i], usum_sh.at[rowv.at[i]],
                             ssem.at[b], add=True)

            @pl.when(i + NBUF3 < NCHUNK)
            def _():
                pltpu.async_copy(t_hbm.at[colv.at[i + NBUF3]],
                                 tvals.at[i + NBUF3], gsem.at[b])
        return 0
    lax.fori_loop(0, NCHUNK // NBUF3, _outer, 0)

    for b in range(NBUF3):
        _scat(0, b).wait()

    plsc.subcore_barrier()
    pltpu.sync_copy(usum_sh.at[pl.ds(s * OPT, OPT)], buf1)
    pltpu.sync_copy(buf1, usum_out.at[pl.ds(c * NPAD + s * OPT, OPT)])


def _phase4_body(s13_ref, up_ref, recip_ref, out_ref):
    u = up_ref[0, :] + up_ref[1, :]
    out_ref[0, :] = jax.nn.sigmoid(s13_ref[0, :] + u * recip_ref[0, :])


_phase4 = pl.pallas_call(
    _phase4_body,
    out_shape=jax.ShapeDtypeStruct((1, N), jnp.float32),
)


def kernel(x, adj, W):
    npad_extra = NPAD - N
    pad = EPAD - E
    row_p = jnp.concatenate(
        [adj[0], N + (jnp.arange(pad, dtype=jnp.int32) % npad_extra)])
    col_p = jnp.concatenate([adj[1], jnp.zeros((pad,), jnp.int32)])
    col3 = col_p.reshape(NW, NCHUNK, CH)
    row3 = row_p.reshape(NW, NCHUNK, CH)
    wr = W.reshape(3, D)
    x0 = x[:, :HD]
    x1 = x[:, HD:]

    xsum_p, deg_p = _phase1(x0, x1, col3, row3)
    xsp = xsum_p.reshape(2, NC, NPAD, HD)
    degp = deg_p.reshape(NC, NPAD)[:, :N]

    t2, s13, recip = _phase2(x, xsp[0, 0, :N], xsp[0, 1, :N],
                             xsp[1, 0, :N], xsp[1, 1, :N], degp, wr)

    (usum_p,) = _phase3(t2.reshape(N), col3, row3)

    score = _phase4(s13, usum_p.reshape(NC, NPAD)[:, :N], recip)
    return score.reshape(N, 1)


# speedup vs baseline: 1.0624x; 1.0624x over previous
import functools

import jax
import jax.numpy as jnp
from jax import lax
from jax.experimental import pallas as pl
from jax.experimental.pallas import tpu as pltpu
from jax.experimental.pallas import tpu_sc as plsc

N = 10000
D = 128
E = 320000
NC, NS = 2, 16
NW = NC * NS
CH = 80
NCHUNK = 128
EPT = NCHUNK * CH
ERT = E // NW
NPAD = 10112
OPT = NPAD // NS
NFULL = OPT // CH
NREM = OPT - NFULL * CH
NBUF = 2
WV = 32
NWAVE = NCHUNK // WV

_mesh = plsc.VectorSubcoreMesh(core_axis_name="c", subcore_axis_name="s")


@functools.partial(
    pl.kernel,
    out_type=[
        jax.ShapeDtypeStruct((NC * NPAD, D), jnp.float32),
        jax.ShapeDtypeStruct((NC * NPAD,), jnp.float32),
    ],
    mesh=_mesh,
    scratch_types=[
        pltpu.VMEM((WV, CH), jnp.int32),
        pltpu.VMEM((WV, CH), jnp.int32),
        pltpu.VMEM((NBUF, CH, D), jnp.float32),
        pltpu.VMEM((CH,), jnp.float32),
        pltpu.VMEM((OPT,), jnp.float32),
        pltpu.VMEM_SHARED((NPAD, D), jnp.float32),
        pltpu.VMEM_SHARED((NPAD,), jnp.float32),
        pltpu.SemaphoreType.DMA((NBUF,)),
        pltpu.SemaphoreType.DMA((NBUF,)),
        pltpu.SemaphoreType.DMA((NBUF,)),
    ],
)
def _phase1(x_hbm, col_hbm, row_hbm, xsum_out, deg_out,
            colv, rowv, rows, onesv, buf1, xsum_sh, deg_sh,
            gsem, ssem, dsem):
    c = lax.axis_index("c")
    s = lax.axis_index("s")
    wid = c * NS + s

    zero16 = jnp.zeros((16,), jnp.float32)
    one16 = jnp.ones((16,), jnp.float32)

    def _z1(i, _):
        buf1[pl.ds(i * 16, 16)] = zero16
        return 0
    lax.fori_loop(0, OPT // 16, _z1, 0)
    buf1[pl.ds(OPT - 16, 16)] = zero16

    def _o1(i, _):
        onesv[pl.ds(i * 16, 16)] = one16
        return 0
    lax.fori_loop(0, CH // 16, _o1, 0)

    def _zrow(i, _):
        for j in range(D // 16):
            rows[0, i, pl.ds(j * 16, 16)] = zero16
        return 0
    lax.fori_loop(0, CH, _zrow, 0)

    for k in range(NFULL):
        pltpu.sync_copy(rows.at[0], xsum_sh.at[pl.ds(s * OPT + k * CH, CH)])
    pltpu.sync_copy(rows.at[0].at[pl.ds(0, NREM)],
                    xsum_sh.at[pl.ds(s * OPT + NFULL * CH, NREM)])
    pltpu.sync_copy(buf1, deg_sh.at[pl.ds(s * OPT, OPT)])
    plsc.subcore_barrier()

    def _gather(j, b):
        return pltpu.make_async_copy(
            x_hbm.at[colv.at[j]], rows.at[b], gsem.at[b])

    def _scat_rows(j, b):
        return pltpu.make_async_copy(
            rows.at[b], xsum_sh.at[rowv.at[j]], ssem.at[b])

    def _scat_deg(j, b):
        return pltpu.make_async_copy(
            onesv, deg_sh.at[rowv.at[j]], dsem.at[b])

    def _wave(w, _):
        pltpu.sync_copy(col_hbm.at[wid, pl.ds(w * WV, WV)], colv)
        pltpu.sync_copy(row_hbm.at[wid, pl.ds(w * WV, WV)], rowv)

        for b in range(NBUF):
            pltpu.async_copy(x_hbm.at[colv.at[b]], rows.at[b], gsem.at[b])

        def _inner(g, _):
            for b in range(NBUF):
                j = g * NBUF + b
                _gather(j, b).wait()

                @pl.when(g > 0)
                def _():
                    _scat_deg(j, b).wait()

                pltpu.async_copy(rows.at[b], xsum_sh.at[rowv.at[j]],
                                 ssem.at[b], add=True)
                pltpu.async_copy(onesv, deg_sh.at[rowv.at[j]],
                                 dsem.at[b], add=True)
                _scat_rows(j, b).wait()

                @pl.when(j + NBUF < WV)
                def _():
                    pltpu.async_copy(x_hbm.at[colv.at[j + NBUF]],
                                     rows.at[b], gsem.at[b])
            return 0
        lax.fori_loop(0, WV // NBUF, _inner, 0)

        for b in range(NBUF):
            _scat_deg(0, b).wait()
        return 0
    lax.fori_loop(0, NWAVE, _wave, 0)

    plsc.subcore_barrier()

    obase = c * NPAD
    for k in range(NFULL):
        off = s * OPT + k * CH
        pltpu.sync_copy(xsum_sh.at[pl.ds(off, CH)], rows.at[0])
        pltpu.sync_copy(rows.at[0], xsum_out.at[pl.ds(obase + off, CH)])
    offr = s * OPT + NFULL * CH
    pltpu.sync_copy(xsum_sh.at[pl.ds(offr, NREM)],
                    rows.at[0].at[pl.ds(0, NREM)])
    pltpu.sync_copy(rows.at[0].at[pl.ds(0, NREM)],
                    xsum_out.at[pl.ds(obase + offr, NREM)])
    pltpu.sync_copy(deg_sh.at[pl.ds(s * OPT, OPT)], buf1)
    pltpu.sync_copy(buf1, deg_out.at[pl.ds(c * NPAD + s * OPT, OPT)])


def _phase2_body(x_ref, xs0_ref, xs1_ref, degp_ref, w_ref,
                 t_ref, s13_ref, recip_ref):
    x = x_ref[...]
    xsum = xs0_ref[...] + xs1_ref[...]
    deg = degp_ref[0, :] + degp_ref[1, :]
    recip = 1.0 / jnp.maximum(deg, 1.0)
    w1 = w_ref[0:1, :]
    w2 = w_ref[1:2, :]
    w3 = w_ref[2:3, :]
    xmean = xsum * recip[:, None]
    t_ref[0, :] = jnp.sum(jnp.abs(x - xmean) * w2, axis=1)
    s13_ref[0, :] = (recip * jnp.sum(xsum * x * w1, axis=1)
                     + jnp.sum(x * w3, axis=1))
    recip_ref[0, :] = recip


_phase2 = pl.pallas_call(
    _phase2_body,
    out_shape=[jax.ShapeDtypeStruct((1, N), jnp.float32)] * 3,
)

NBUF3 = 8


@functools.partial(
    pl.kernel,
    out_type=[jax.ShapeDtypeStruct((NC * NPAD,), jnp.float32)],
    mesh=_mesh,
    scratch_types=[
        pltpu.VMEM((NCHUNK, CH), jnp.int32),
        pltpu.VMEM((NCHUNK, CH), jnp.int32),
        pltpu.VMEM((NCHUNK, CH), jnp.float32),
        pltpu.VMEM((OPT,), jnp.float32),
        pltpu.VMEM_SHARED((NPAD,), jnp.float32),
        pltpu.SemaphoreType.DMA((NBUF3,)),
        pltpu.SemaphoreType.DMA((NBUF3,)),
    ],
)
def _phase3(t_hbm, col_hbm, row_hbm, usum_out,
            colv, rowv, tvals, buf1, usum_sh, gsem, ssem):
    c = lax.axis_index("c")
    s = lax.axis_index("s")
    wid = c * NS + s

    zero16 = jnp.zeros((16,), jnp.float32)

    def _z1(i, _):
        buf1[pl.ds(i * 16, 16)] = zero16
        return 0
    lax.fori_loop(0, OPT // 16, _z1, 0)
    buf1[pl.ds(OPT - 16, 16)] = zero16
    pltpu.sync_copy(buf1, usum_sh.at[pl.ds(s * OPT, OPT)])

    pltpu.sync_copy(col_hbm.at[wid], colv)
    pltpu.sync_copy(row_hbm.at[wid], rowv)
    plsc.subcore_barrier()

    def _gather(i, b):
        return pltpu.make_async_copy(
            t_hbm.at[colv.at[i]], tvals.at[i], gsem.at[b])

    def _scat(i, b):
        return pltpu.make_async_copy(
            tvals.at[i], usum_sh.at[rowv.at[i]], ssem.at[b])

    for b in range(NBUF3):
        pltpu.async_copy(t_hbm.at[colv.at[b]], tvals.at[b], gsem.at[b])

    def _outer(g, _):
        for b in range(NBUF3):
            i = g * NBUF3 + b
            _gather(i, b).wait()

            @pl.when(g > 0)
            def _():
                _scat(i, b).wait()

            pltpu.async_copy(tvals.at[i], usum_sh.at[rowv.at[i]],
                             ssem.at[b], add=True)

            @pl.when(i + NBUF3 < NCHUNK)
            def _():
                pltpu.async_copy(t_hbm.at[colv.at[i + NBUF3]],
                                 tvals.at[i + NBUF3], gsem.at[b])
        return 0
    lax.fori_loop(0, NCHUNK // NBUF3, _outer, 0)

    for b in range(NBUF3):
        _scat(0, b).wait()

    plsc.subcore_barrier()
    pltpu.sync_copy(usum_sh.at[pl.ds(s * OPT, OPT)], buf1)
    pltpu.sync_copy(buf1, usum_out.at[pl.ds(c * NPAD + s * OPT, OPT)])


def _phase4_body(s13_ref, up_ref, recip_ref, out_ref):
    u = up_ref[0, :] + up_ref[1, :]
    out_ref[0, :] = jax.nn.sigmoid(s13_ref[0, :] + u * recip_ref[0, :])


_phase4 = pl.pallas_call(
    _phase4_body,
    out_shape=jax.ShapeDtypeStruct((1, N), jnp.float32),
)


def kernel(x, adj, W):
    npad_extra = NPAD - N
    ppt = EPT - ERT
    pad_rows = N + (jnp.arange(NW * ppt, dtype=jnp.int32) % npad_extra)
    row_p = jnp.concatenate(
        [adj[0].reshape(NW, ERT), pad_rows.reshape(NW, ppt)], axis=1)
    col_p = jnp.concatenate(
        [adj[1].reshape(NW, ERT),
         jnp.zeros((NW, ppt), jnp.int32)], axis=1)
    col3 = col_p.reshape(NW, NCHUNK, CH)
    row3 = row_p.reshape(NW, NCHUNK, CH)
    wr = W.reshape(3, D)

    xsum_p, deg_p = _phase1(x, col3, row3)
    xsp = xsum_p.reshape(NC, NPAD, D)
    degp = deg_p.reshape(NC, NPAD)[:, :N]

    t2, s13, recip = _phase2(x, xsp[0, :N], xsp[1, :N], degp, wr)

    (usum_p,) = _phase3(t2.reshape(N), col3, row3)

    score = _phase4(s13, usum_p.reshape(NC, NPAD)[:, :N], recip)
    return score.reshape(N, 1)
